# baseline (speedup 1.0000x reference)
import functools

import jax
import jax.numpy as jnp
from jax import lax
from jax.experimental import pallas as pl
from jax.experimental.pallas import tpu as pltpu
from jax.experimental.pallas import tpu_sc as plsc

_VOCAB = 1000000
_EMB = 32
_TAGS = 50
_BATCH = 16384

_NC = 2
_NS = 16
_NW = _NC * _NS
_BPW = _BATCH // _NW
_CHUNK = 128
_NCHUNK = _BPW // _CHUNK

_sc_mesh = plsc.VectorSubcoreMesh(core_axis_name="c", subcore_axis_name="s")


@functools.partial(
    pl.kernel,
    mesh=_sc_mesh,
    out_type=jax.ShapeDtypeStruct((_BATCH, _EMB), jnp.float32),
    scratch_types=[
        pltpu.VMEM((_NCHUNK, _CHUNK), jnp.int32),
        pltpu.VMEM((_BPW, _EMB), jnp.float32),
        pltpu.SemaphoreType.DMA,
    ],
    compiler_params=pltpu.CompilerParams(use_tc_tiling_on_sc=False),
)
def _sc_gather(tokens_hbm, table_hbm, out_hbm, idx_v, rows_v, sem):
    wid = lax.axis_index("s") * _NC + lax.axis_index("c")
    base = wid * _BPW
    pltpu.sync_copy(tokens_hbm.at[pl.ds(wid * _NCHUNK, _NCHUNK)], idx_v)
    copies = []
    for j in range(_NCHUNK):
        copies.append(
            pltpu.async_copy(
                table_hbm.at[idx_v.at[j]],
                rows_v.at[pl.ds(j * _CHUNK, _CHUNK)],
                sem,
            )
        )
    for c in copies:
        c.wait()
    pltpu.sync_copy(rows_v, out_hbm.at[pl.ds(base, _BPW)])


_TC_BLK = 2048


def _tc_body(e_ref, w_ref, b_ref, o_ref):
    e = e_ref[...]
    w = w_ref[...]
    f = lax.dot_general(
        e, w, (((1,), (1,)), ((), ())),
        preferred_element_type=jnp.float32,
        precision=lax.Precision.HIGHEST,
    ) + b_ref[...]
    m = jnp.max(f, axis=-1, keepdims=True)
    s = f - m
    o_ref[...] = s - jnp.log(jnp.sum(jnp.exp(s), axis=-1, keepdims=True))


_tc_tail = pl.pallas_call(
    _tc_body,
    grid=(_BATCH // _TC_BLK,),
    in_specs=[
        pl.BlockSpec((_TC_BLK, _EMB), lambda i: (i, 0)),
        pl.BlockSpec((_TAGS, _EMB), lambda i: (0, 0)),
        pl.BlockSpec((1, _TAGS), lambda i: (0, 0)),
    ],
    out_specs=pl.BlockSpec((_TC_BLK, _TAGS), lambda i: (i, 0)),
    out_shape=jax.ShapeDtypeStruct((_BATCH, _TAGS), jnp.float32),
    compiler_params=pltpu.CompilerParams(
        dimension_semantics=("parallel",),
    ),
)


def kernel(tokens, emb_table, W, b):
    tok2d = tokens.astype(jnp.int32).reshape(_NW * _NCHUNK, _CHUNK)
    e = _sc_gather(tok2d, emb_table)
    return _tc_tail(e, W, b.reshape(1, _TAGS))
